# baseline (speedup 1.0000x reference)
import functools

import jax
import jax.numpy as jnp
from jax import lax
from jax.experimental import pallas as pl
from jax.experimental.pallas import tpu as pltpu
from jax.experimental.pallas import tpu_sc as plsc

NW = 32
GK = 16
LANES = 16


def _ds16(c):
    return pl.ds(c * 16, 16)


def _lanes_sum(v):
    lane = lax.iota(jnp.int32, LANES)
    for k in (1, 2, 4, 8):
        v = v + v.at[lane ^ k].get(mode="promise_in_bounds")
    return v


def _stats_body(h_ref, o_ref):
    i = pl.program_id(0)
    blk = h_ref[...]
    s = jnp.sum(blk, axis=0, keepdims=True)
    q = jnp.sum(blk * blk, axis=0, keepdims=True)

    @pl.when(i == 0)
    def _():
        o_ref[...] = jnp.zeros_like(o_ref)

    o_ref[0:1, :] += s
    o_ref[1:2, :] += q


def _col_stats(h):
    m, f = h.shape
    mb = 400
    return pl.pallas_call(
        _stats_body,
        out_shape=jax.ShapeDtypeStruct((8, f), jnp.float32),
        grid=(m // mb,),
        in_specs=[pl.BlockSpec((mb, f), lambda i: (i, 0))],
        out_specs=pl.BlockSpec((8, f), lambda i: (0, 0)),
    )(h)


def _mm_body(relu, sig, h_ref, w_ref, s_ref, t_ref, b_ref, o_ref):
    a = h_ref[...] * s_ref[...] + t_ref[...]
    if relu:
        a = jnp.maximum(a, 0.0)
    o = jnp.dot(a, w_ref[...], preferred_element_type=jnp.float32)
    o = o + b_ref[...]
    if sig:
        o = jax.nn.sigmoid(o)
    o_ref[...] = o


def _affine_matmul(h, w, s, t, bias, relu=False, sig=False):
    m, f = h.shape
    q = w.shape[1]
    mb = 400
    body = functools.partial(_mm_body, relu, sig)
    return pl.pallas_call(
        body,
        out_shape=jax.ShapeDtypeStruct((m, q), jnp.float32),
        grid=(m // mb,),
        in_specs=[
            pl.BlockSpec((mb, f), lambda i: (i, 0)),
            pl.BlockSpec((f, q), lambda i: (0, 0)),
            pl.BlockSpec((1, f), lambda i: (0, 0)),
            pl.BlockSpec((1, f), lambda i: (0, 0)),
            pl.BlockSpec((1, q), lambda i: (0, 0)),
        ],
        out_specs=pl.BlockSpec((mb, q), lambda i: (i, 0)),
    )(h, w, s.reshape(1, f), t.reshape(1, f), bias.reshape(1, q))


def _gat_sc(xl, xr, att_flat, bias, s_src, s_dst, ws, n, npw, h_heads, c_dim,
            gk):
    hc = h_heads * c_dim
    nchunk = hc // LANES
    per_head = c_dim // LANES
    mesh = plsc.VectorSubcoreMesh(core_axis_name="c", subcore_axis_name="s")

    @functools.partial(
        pl.kernel,
        mesh=mesh,
        out_type=jax.ShapeDtypeStruct((n * hc,), jnp.float32),
        scratch_types=[
            pltpu.VMEM((gk,), jnp.int32),
            pltpu.VMEM((gk,), jnp.int32),
            pltpu.VMEM((gk + LANES,), jnp.int32),
            pltpu.VMEM((gk + LANES,), jnp.int32),
            pltpu.VMEM((gk, hc), jnp.float32),
            pltpu.VMEM((gk, hc), jnp.float32),
            pltpu.VMEM((2, hc), jnp.float32),
            pltpu.VMEM((hc,), jnp.float32),
            pltpu.VMEM((hc,), jnp.float32),
            pltpu.VMEM((hc,), jnp.float32),
            pltpu.VMEM((4 * hc,), jnp.float32),
            pltpu.VMEM((56,), jnp.int32),
            pltpu.SemaphoreType.DMA,
            pltpu.SemaphoreType.DMA,
            pltpu.SemaphoreType.DMA,
            pltpu.SemaphoreType.DMA,
            pltpu.SemaphoreType.DMA,
            pltpu.SemaphoreType.DMA,
        ],
    )
    def k(xl_h, xr_h, att_h, bias_h, src_h, dst_h, ws_h, out_h,
          idxs0, idxs1, idxd0, idxd1, rows0, rows1, xr_v, att_v, bias_v,
          acc_v, ob_v, ws_v, sem_g0, sem_g1, sem_i0, sem_i1, sem_x, sem_o):
        idxs = (idxs0, idxs1)
        idxd = (idxd0, idxd1)
        rows = (rows0, rows1)
        sem_g = (sem_g0, sem_g1)
        sem_i = (sem_i0, sem_i1)

        wid = lax.axis_index("s") * 2 + lax.axis_index("c")
        pltpu.sync_copy(ws_h, ws_v)
        pltpu.sync_copy(att_h, att_v)
        pltpu.sync_copy(bias_h, bias_v)
        wsv = ws_v[pl.ds(wid, LANES)]
        e0 = wsv[0]
        e1 = wsv[1]
        n0 = wid * npw
        n1 = jnp.minimum(n0 + npw, n)
        a0 = pl.multiple_of((e0 >> 3) << 3, 8)
        ng = (e1 - a0 + gk - 1) // gk

        for c in range(nchunk):
            acc_v[_ds16(c)] = jnp.zeros((LANES,), jnp.float32)

        zero16 = jnp.zeros((LANES,), jnp.float32)

        def idx_copies(g, b):
            base = pl.multiple_of(a0 + g * gk, 8)
            return (pltpu.make_async_copy(src_h.at[pl.ds(base, gk)],
                                          idxs[b], sem_i[b]),
                    pltpu.make_async_copy(dst_h.at[pl.ds(base, gk)],
                                          idxd[b].at[pl.ds(0, gk)],
                                          sem_i[b]))

        def idx_start(g, b):
            for cp in idx_copies(g, b):
                cp.start()

        def idx_wait(g, b):
            for cp in idx_copies(g, b):
                cp.wait()

        def gather(b):
            return pltpu.make_async_copy(xl_h.at[idxs[b]], rows[b], sem_g[b])

        def pair_copy(pq, row0):
            return pltpu.make_async_copy(
                ob_v.at[pl.ds(pl.multiple_of(pq * (2 * hc), 8), 2 * hc)],
                out_h.at[pl.ds(pl.multiple_of(row0 * hc, 8), 2 * hc)],
                sem_o)

        def finalize(cur_d, dsums, ecnt, last):
            kk = cur_d - n0
            pq = (kk >> 1) & 1
            soff = (pq * 2 + (kk & 1)) * hc
            for h in range(h_heads):
                qv = 1.0 / (dsums[h] * ecnt)
                for c in range(h * per_head, (h + 1) * per_head):
                    ob_v[pl.ds(soff + c * LANES, LANES)] = (
                        acc_v[_ds16(c)] * qv + bias_v[_ds16(c)])
            for c in range(nchunk):
                acc_v[_ds16(c)] = jnp.zeros((LANES,), jnp.float32)

            @pl.when((kk & 1) == 1)
            def _():
                @pl.when(kk >= 3)
                def _():
                    pair_copy(pq, cur_d - 1).wait()
                pair_copy(pq, cur_d - 1).start()

            if last:
                @pl.when((kk & 1) == 0)
                def _():
                    @pl.when(kk >= 2)
                    def _():
                        pair_copy(pq, cur_d - 2).wait()
                    single = pltpu.make_async_copy(
                        ob_v.at[pl.ds(pl.multiple_of(soff, 8), hc)],
                        out_h.at[pl.ds(pl.multiple_of(cur_d * hc, 8), hc)],
                        sem_o)
                    single.start()
                    single.wait()

                @pl.when((kk & 1) == 1)
                def _():
                    pair_copy(pq, cur_d - 1).wait()

        def apply_prev(rref, jprev, wm):
            for h in range(h_heads):
                for c in range(h * per_head, (h + 1) * per_head):
                    plsc.addupdate(acc_v.at[_ds16(c)],
                                   wm[h] * rref[jprev, _ds16(c)])

        def make_inner(b, base):
            def inner(j, c2):
                cur_d, dsums, ecnt, wm, em = c2
                e = base + j
                active = jnp.logical_and(e >= e0, e < e1)
                d = idxd[b][pl.ds(j, LANES)][0]
                newseg = jnp.logical_and(active, d != cur_d)
                dofin = jnp.logical_and(newseg, cur_d >= 0)
                jm1 = jnp.maximum(j - 1, 0)

                dsums1 = tuple(dsums[h] + wm[h] for h in range(h_heads))
                ecnt1 = ecnt + em

                @pl.when(dofin)
                def _():
                    apply_prev(rows[b], jm1, wm)
                    finalize(cur_d, dsums1, ecnt1, last=False)

                @pl.when(newseg)
                def _():
                    pd = (d - n0) & 1
                    pltpu.make_async_copy(
                        xr_h.at[pl.ds(pl.multiple_of(d * hc, 8), hc)],
                        xr_v.at[pd], sem_x).wait()

                    @pl.when(d + 1 < n1)
                    def _():
                        pltpu.make_async_copy(
                            xr_h.at[pl.ds(pl.multiple_of((d + 1) * hc, 8), hc)],
                            xr_v.at[1 - pd], sem_x).start()

                cur_d2 = jnp.where(newseg, d, cur_d)
                pcur = (cur_d2 - n0) & 1
                wmA = [jnp.where(dofin, zero16, wm[h]) for h in range(h_heads)]
                apply_prev(rows[b], jm1, wmA)

                wvecs = []
                for h in range(h_heads):
                    daccs = [zero16] * 4
                    for ci, c in enumerate(range(h * per_head,
                                                 (h + 1) * per_head)):
                        t = rows[b][j, _ds16(c)] + xr_v[pcur, _ds16(c)]
                        lr = jnp.maximum(t, 0.2 * t)
                        daccs[ci & 3] = daccs[ci & 3] + lr * att_v[_ds16(c)]
                    dacc = (daccs[0] + daccs[1]) + (daccs[2] + daccs[3])
                    wvecs.append(jnp.exp(_lanes_sum(dacc)))

                dsums2 = tuple(jnp.where(newseg, zero16, dsums1[h])
                               for h in range(h_heads))
                ecnt2 = jnp.where(newseg, zero16, ecnt1)
                wm2 = tuple(jnp.where(active, wvecs[h], zero16)
                            for h in range(h_heads))
                em2 = jnp.where(active, jnp.full((LANES,), 1.0), zero16)
                return (cur_d2, dsums2, ecnt2, wm2, em2)

            return inner

        idx_start(0, 0)
        idx_wait(0, 0)
        gather(0).start()

        @pl.when(1 < ng)
        def _():
            idx_start(1, 1)

        pltpu.make_async_copy(xr_h.at[pl.ds(pl.multiple_of(n0 * hc, 8), hc)],
                              xr_v.at[0], sem_x).start()

        def big(gg, carry):
            for b in (0, 1):
                g = 2 * gg + b
                ok = g < ng
                cur_d, dsums, ecnt, wm, em = carry
                apply_prev(rows[1 - b], gk - 1, wm)
                dsums = tuple(dsums[h] + wm[h] for h in range(h_heads))
                ecnt = ecnt + em
                carry = (cur_d, dsums, ecnt,
                         tuple(zero16 for _ in range(h_heads)), zero16)

                @pl.when(ok)
                def _():
                    gather(b).wait()

                @pl.when(g + 1 < ng)
                def _():
                    idx_wait(g + 1, 1 - b)
                    gather(1 - b).start()

                base = pl.multiple_of(a0 + g * gk, 8)
                trip = jnp.where(ok, gk, 0)
                carry = lax.fori_loop(0, trip, make_inner(b, base), carry)

                @pl.when(g + 2 < ng)
                def _():
                    idx_start(g + 2, b)
            return carry

        carry0 = (jnp.int32(-1), tuple(zero16 for _ in range(h_heads)),
                  zero16, tuple(zero16 for _ in range(h_heads)), zero16)
        cur_d, dsums, ecnt, wm, em = lax.fori_loop(0, (ng + 1) // 2, big,
                                                   carry0)

        @pl.when((ng & 1) == 0)
        def _():
            apply_prev(rows[1], gk - 1, wm)

        dsums = tuple(dsums[h] + wm[h] for h in range(h_heads))
        ecnt = ecnt + em

        @pl.when(cur_d >= 0)
        def _():
            finalize(cur_d, dsums, ecnt, last=True)

    return k(xl, xr.reshape(-1), att_flat, bias, s_src, s_dst,
             ws).reshape(n, hc)


def _bn_affine(stats, g, b, m):
    mu = stats[0] / m
    var = stats[1] / m - mu * mu
    s = g * lax.rsqrt(var + 1e-5)
    return s, b - mu * s


def kernel(x, edge_index, y, train_mask, bn0_g, bn0_b, W1l, W1r, a1, b1,
           bn1_g, bn1_b, W2l, W2r, a2, b2, bn2_g, bn2_b, W3l, W3r, a3, b3,
           bn3_g, bn3_b, W4l, W4r, a4, b4, lin_W, lin_b):
    n = x.shape[0]
    e = edge_index.shape[1]
    e2 = e + n

    loop = jnp.arange(n, dtype=jnp.int32)
    src = jnp.concatenate([edge_index[0].astype(jnp.int32), loop])
    dst = jnp.concatenate([edge_index[1].astype(jnp.int32), loop])
    order = jnp.argsort(dst)
    s_src = src[order]
    s_dst = dst[order]
    npw = ((n + NW - 1) // NW + 7) // 8 * 8
    wnodes = jnp.minimum(jnp.arange(NW + 1, dtype=jnp.int32) * npw, n)
    wstarts = jnp.sum((dst[None, :] < wnodes[:, None]).astype(jnp.int32),
                      axis=1)
    ws = jnp.zeros((56,), jnp.int32).at[: NW + 1].set(wstarts)
    pad = 64
    s_src = jnp.concatenate([s_src, jnp.zeros((pad,), jnp.int32)])
    s_dst = jnp.concatenate([s_dst, jnp.full((pad,), -1, jnp.int32)])

    def gat_layer(h, g, b, wl, wr, att, bias, heads, cdim, relu):
        f = h.shape[1]
        stats = _col_stats(h)
        s, t = _bn_affine(stats, g, b, jnp.float32(n))
        wcat = jnp.concatenate([wl, wr], axis=1)
        zq = jnp.zeros((wcat.shape[1],), jnp.float32)
        xlr = _affine_matmul(h, wcat, s, t, zq, relu=relu)
        hc = heads * cdim
        xl, xr = xlr[:, :hc], xlr[:, hc:]
        gk = 16 if hc > 512 else 32
        return _gat_sc(xl, xr, att.reshape(hc), bias, s_src, s_dst, ws,
                       n, npw, heads, cdim, gk)

    h = gat_layer(x, bn0_g, bn0_b, W1l, W1r, a1, b1, 2, 512, relu=False)
    h = gat_layer(h, bn1_g, bn1_b, W2l, W2r, a2, b2, 1, 512, relu=True)
    h = gat_layer(h, bn2_g, bn2_b, W3l, W3r, a3, b3, 1, 512, relu=True)
    h = gat_layer(h, bn3_g, bn3_b, W4l, W4r, a4, b4, 1, 512, relu=True)

    ones = jnp.ones((h.shape[1],), jnp.float32)
    zeros = jnp.zeros((h.shape[1],), jnp.float32)
    yp = _affine_matmul(h, lin_W, ones, zeros, lin_b, relu=True, sig=True)
    y_pred = yp[:, 0]

    idx = jnp.nonzero(train_mask, size=train_mask.shape[0], fill_value=0)[0]
    return (y_pred[idx], y[idx])


# speedup vs baseline: 1.0236x; 1.0236x over previous
import functools

import jax
import jax.numpy as jnp
from jax import lax
from jax.experimental import pallas as pl
from jax.experimental.pallas import tpu as pltpu
from jax.experimental.pallas import tpu_sc as plsc

NW = 32
GK = 16
LANES = 16


def _ds16(c):
    return pl.ds(c * 16, 16)


def _lanes_sum(v):
    lane = lax.iota(jnp.int32, LANES)
    for k in (1, 2, 4, 8):
        v = v + v.at[lane ^ k].get(mode="promise_in_bounds")
    return v


def _stats_body(h_ref, o_ref):
    i = pl.program_id(0)
    blk = h_ref[...]
    s = jnp.sum(blk, axis=0, keepdims=True)
    q = jnp.sum(blk * blk, axis=0, keepdims=True)

    @pl.when(i == 0)
    def _():
        o_ref[...] = jnp.zeros_like(o_ref)

    o_ref[0:1, :] += s
    o_ref[1:2, :] += q


def _col_stats(h):
    m, f = h.shape
    mb = 400
    return pl.pallas_call(
        _stats_body,
        out_shape=jax.ShapeDtypeStruct((8, f), jnp.float32),
        grid=(m // mb,),
        in_specs=[pl.BlockSpec((mb, f), lambda i: (i, 0))],
        out_specs=pl.BlockSpec((8, f), lambda i: (0, 0)),
    )(h)


def _mm_body(relu, sig, h_ref, w_ref, s_ref, t_ref, b_ref, o_ref):
    a = h_ref[...] * s_ref[...] + t_ref[...]
    if relu:
        a = jnp.maximum(a, 0.0)
    o = jnp.dot(a, w_ref[...], preferred_element_type=jnp.float32)
    o = o + b_ref[...]
    if sig:
        o = jax.nn.sigmoid(o)
    o_ref[...] = o


def _affine_matmul(h, w, s, t, bias, relu=False, sig=False):
    m, f = h.shape
    q = w.shape[1]
    mb = 400
    body = functools.partial(_mm_body, relu, sig)
    return pl.pallas_call(
        body,
        out_shape=jax.ShapeDtypeStruct((m, q), jnp.float32),
        grid=(m // mb,),
        in_specs=[
            pl.BlockSpec((mb, f), lambda i: (i, 0)),
            pl.BlockSpec((f, q), lambda i: (0, 0)),
            pl.BlockSpec((1, f), lambda i: (0, 0)),
            pl.BlockSpec((1, f), lambda i: (0, 0)),
            pl.BlockSpec((1, q), lambda i: (0, 0)),
        ],
        out_specs=pl.BlockSpec((mb, q), lambda i: (i, 0)),
    )(h, w, s.reshape(1, f), t.reshape(1, f), bias.reshape(1, q))


def _gat_sc(xl, xr, att_flat, bias, s_src, s_dst, ws, n, npw, h_heads, c_dim,
            gk):
    hc = h_heads * c_dim
    nchunk = hc // LANES
    per_head = c_dim // LANES
    mesh = plsc.VectorSubcoreMesh(core_axis_name="c", subcore_axis_name="s")

    @functools.partial(
        pl.kernel,
        mesh=mesh,
        out_type=jax.ShapeDtypeStruct((n * hc,), jnp.float32),
        scratch_types=[
            pltpu.VMEM((gk,), jnp.int32),
            pltpu.VMEM((gk,), jnp.int32),
            pltpu.VMEM((gk + LANES,), jnp.int32),
            pltpu.VMEM((gk + LANES,), jnp.int32),
            pltpu.VMEM((gk, hc), jnp.float32),
            pltpu.VMEM((gk, hc), jnp.float32),
            pltpu.VMEM((2, hc), jnp.float32),
            pltpu.VMEM((hc,), jnp.float32),
            pltpu.VMEM((hc,), jnp.float32),
            pltpu.VMEM((hc,), jnp.float32),
            pltpu.VMEM((4 * hc,), jnp.float32),
            pltpu.VMEM((56,), jnp.int32),
            pltpu.SemaphoreType.DMA,
            pltpu.SemaphoreType.DMA,
            pltpu.SemaphoreType.DMA,
            pltpu.SemaphoreType.DMA,
            pltpu.SemaphoreType.DMA,
            pltpu.SemaphoreType.DMA,
        ],
    )
    def k(xl_h, xr_h, att_h, bias_h, src_h, dst_h, ws_h, out_h,
          idxs0, idxs1, idxd0, idxd1, rows0, rows1, xr_v, att_v, bias_v,
          acc_v, ob_v, ws_v, sem_g0, sem_g1, sem_i0, sem_i1, sem_x, sem_o):
        idxs = (idxs0, idxs1)
        idxd = (idxd0, idxd1)
        rows = (rows0, rows1)
        sem_g = (sem_g0, sem_g1)
        sem_i = (sem_i0, sem_i1)

        wid = lax.axis_index("s") * 2 + lax.axis_index("c")
        pltpu.sync_copy(ws_h, ws_v)
        pltpu.sync_copy(att_h, att_v)
        pltpu.sync_copy(bias_h, bias_v)
        wsv = ws_v[pl.ds(wid, LANES)]
        e0 = wsv[0]
        e1 = wsv[1]
        n0 = wid * npw
        n1 = jnp.minimum(n0 + npw, n)
        a0 = pl.multiple_of((e0 >> 3) << 3, 8)
        ng = (e1 - a0 + gk - 1) // gk

        for c in range(nchunk):
            acc_v[_ds16(c)] = jnp.zeros((LANES,), jnp.float32)

        zero16 = jnp.zeros((LANES,), jnp.float32)

        def idx_copies(g, b):
            base = pl.multiple_of(a0 + g * gk, 8)
            return (pltpu.make_async_copy(src_h.at[pl.ds(base, gk)],
                                          idxs[b], sem_i[b]),
                    pltpu.make_async_copy(dst_h.at[pl.ds(base, gk)],
                                          idxd[b].at[pl.ds(0, gk)],
                                          sem_i[b]))

        def idx_start(g, b):
            for cp in idx_copies(g, b):
                cp.start()

        def idx_wait(g, b):
            for cp in idx_copies(g, b):
                cp.wait()

        def gather(b):
            return pltpu.make_async_copy(xl_h.at[idxs[b]], rows[b], sem_g[b])

        def pair_copy(pq, row0):
            return pltpu.make_async_copy(
                ob_v.at[pl.ds(pl.multiple_of(pq * (2 * hc), 8), 2 * hc)],
                out_h.at[pl.ds(pl.multiple_of(row0 * hc, 8), 2 * hc)],
                sem_o)

        def finalize(cur_d, dsums, ecnt, last):
            kk = cur_d - n0
            pq = (kk >> 1) & 1
            soff = (pq * 2 + (kk & 1)) * hc
            for h in range(h_heads):
                qv = 1.0 / (dsums[h] * ecnt)
                for c in range(h * per_head, (h + 1) * per_head):
                    ob_v[pl.ds(soff + c * LANES, LANES)] = (
                        acc_v[_ds16(c)] * qv + bias_v[_ds16(c)])
            for c in range(nchunk):
                acc_v[_ds16(c)] = jnp.zeros((LANES,), jnp.float32)

            @pl.when((kk & 1) == 1)
            def _():
                @pl.when(kk >= 3)
                def _():
                    pair_copy(pq, cur_d - 1).wait()
                pair_copy(pq, cur_d - 1).start()

            if last:
                @pl.when((kk & 1) == 0)
                def _():
                    @pl.when(kk >= 2)
                    def _():
                        pair_copy(pq, cur_d - 2).wait()
                    single = pltpu.make_async_copy(
                        ob_v.at[pl.ds(pl.multiple_of(soff, 8), hc)],
                        out_h.at[pl.ds(pl.multiple_of(cur_d * hc, 8), hc)],
                        sem_o)
                    single.start()
                    single.wait()

                @pl.when((kk & 1) == 1)
                def _():
                    pair_copy(pq, cur_d - 1).wait()

        def apply_prev(rref, jprev, wm):
            for h in range(h_heads):
                for c in range(h * per_head, (h + 1) * per_head):
                    plsc.addupdate(acc_v.at[_ds16(c)],
                                   wm[h] * rref[jprev, _ds16(c)])

        def make_inner(b, base):
            def inner(j, c2):
                cur_d, dsums, ecnt, wm, em = c2
                e = base + j
                active = jnp.logical_and(e >= e0, e < e1)
                d = idxd[b][pl.ds(j, LANES)][0]
                newseg = jnp.logical_and(active, d != cur_d)
                dofin = jnp.logical_and(newseg, cur_d >= 0)
                jm1 = jnp.maximum(j - 1, 0)

                dsums1 = tuple(dsums[h] + wm[h] for h in range(h_heads))
                ecnt1 = ecnt + em

                @pl.when(dofin)
                def _():
                    apply_prev(rows[b], jm1, wm)
                    finalize(cur_d, dsums1, ecnt1, last=False)

                @pl.when(newseg)
                def _():
                    pd = (d - n0) & 1
                    pltpu.make_async_copy(
                        xr_h.at[pl.ds(pl.multiple_of(d * hc, 8), hc)],
                        xr_v.at[pd], sem_x).wait()

                    @pl.when(d + 1 < n1)
                    def _():
                        pltpu.make_async_copy(
                            xr_h.at[pl.ds(pl.multiple_of((d + 1) * hc, 8), hc)],
                            xr_v.at[1 - pd], sem_x).start()

                cur_d2 = jnp.where(newseg, d, cur_d)
                pcur = (cur_d2 - n0) & 1
                wmA = [jnp.where(dofin, zero16, wm[h]) for h in range(h_heads)]
                apply_prev(rows[b], jm1, wmA)

                wvecs = []
                for h in range(h_heads):
                    daccs = [zero16] * 4
                    for ci, c in enumerate(range(h * per_head,
                                                 (h + 1) * per_head)):
                        t = rows[b][j, _ds16(c)] + xr_v[pcur, _ds16(c)]
                        lr = jnp.maximum(t, 0.2 * t)
                        daccs[ci & 3] = daccs[ci & 3] + lr * att_v[_ds16(c)]
                    dacc = (daccs[0] + daccs[1]) + (daccs[2] + daccs[3])
                    wvecs.append(jnp.exp(_lanes_sum(dacc)))

                dsums2 = tuple(jnp.where(newseg, zero16, dsums1[h])
                               for h in range(h_heads))
                ecnt2 = jnp.where(newseg, zero16, ecnt1)
                wm2 = tuple(jnp.where(active, wvecs[h], zero16)
                            for h in range(h_heads))
                em2 = jnp.where(active, jnp.full((LANES,), 1.0), zero16)
                return (cur_d2, dsums2, ecnt2, wm2, em2)

            return inner

        idx_start(0, 0)
        idx_wait(0, 0)
        gather(0).start()

        @pl.when(1 < ng)
        def _():
            idx_start(1, 1)

        pltpu.make_async_copy(xr_h.at[pl.ds(pl.multiple_of(n0 * hc, 8), hc)],
                              xr_v.at[0], sem_x).start()

        def big(gg, carry):
            for b in (0, 1):
                g = 2 * gg + b
                ok = g < ng
                cur_d, dsums, ecnt, wm, em = carry
                apply_prev(rows[1 - b], gk - 1, wm)
                dsums = tuple(dsums[h] + wm[h] for h in range(h_heads))
                ecnt = ecnt + em
                carry = (cur_d, dsums, ecnt,
                         tuple(zero16 for _ in range(h_heads)), zero16)

                @pl.when(ok)
                def _():
                    gather(b).wait()

                @pl.when(g + 1 < ng)
                def _():
                    idx_wait(g + 1, 1 - b)
                    gather(1 - b).start()

                base = pl.multiple_of(a0 + g * gk, 8)
                trip = jnp.where(ok, gk, 0)
                carry = lax.fori_loop(0, trip, make_inner(b, base), carry)

                @pl.when(g + 2 < ng)
                def _():
                    idx_start(g + 2, b)
            return carry

        carry0 = (jnp.int32(-1), tuple(zero16 for _ in range(h_heads)),
                  zero16, tuple(zero16 for _ in range(h_heads)), zero16)
        cur_d, dsums, ecnt, wm, em = lax.fori_loop(0, (ng + 1) // 2, big,
                                                   carry0)

        @pl.when((ng & 1) == 0)
        def _():
            apply_prev(rows[1], gk - 1, wm)

        dsums = tuple(dsums[h] + wm[h] for h in range(h_heads))
        ecnt = ecnt + em

        @pl.when(cur_d >= 0)
        def _():
            finalize(cur_d, dsums, ecnt, last=True)

    return k(xl, xr.reshape(-1), att_flat, bias, s_src, s_dst,
             ws).reshape(n, hc)


def _bn_affine(stats, g, b, m):
    mu = stats[0] / m
    var = stats[1] / m - mu * mu
    s = g * lax.rsqrt(var + 1e-5)
    return s, b - mu * s


def kernel(x, edge_index, y, train_mask, bn0_g, bn0_b, W1l, W1r, a1, b1,
           bn1_g, bn1_b, W2l, W2r, a2, b2, bn2_g, bn2_b, W3l, W3r, a3, b3,
           bn3_g, bn3_b, W4l, W4r, a4, b4, lin_W, lin_b):
    n = x.shape[0]
    e = edge_index.shape[1]
    e2 = e + n

    loop = jnp.arange(n, dtype=jnp.int32)
    src = jnp.concatenate([edge_index[0].astype(jnp.int32), loop])
    dst = jnp.concatenate([edge_index[1].astype(jnp.int32), loop])
    order = jnp.argsort(dst)
    s_src = src[order]
    s_dst = dst[order]
    npw = ((n + NW - 1) // NW + 7) // 8 * 8
    wnodes = jnp.minimum(jnp.arange(NW + 1, dtype=jnp.int32) * npw, n)
    wstarts = jnp.sum((dst[None, :] < wnodes[:, None]).astype(jnp.int32),
                      axis=1)
    ws = jnp.zeros((56,), jnp.int32).at[: NW + 1].set(wstarts)
    pad = 64
    s_src = jnp.concatenate([s_src, jnp.zeros((pad,), jnp.int32)])
    s_dst = jnp.concatenate([s_dst, jnp.full((pad,), -1, jnp.int32)])

    def gat_layer(h, g, b, wl, wr, att, bias, heads, cdim, relu):
        f = h.shape[1]
        stats = _col_stats(h)
        s, t = _bn_affine(stats, g, b, jnp.float32(n))
        wcat = jnp.concatenate([wl, wr], axis=1)
        zq = jnp.zeros((wcat.shape[1],), jnp.float32)
        xlr = _affine_matmul(h, wcat, s, t, zq, relu=relu)
        hc = heads * cdim
        xl, xr = xlr[:, :hc], xlr[:, hc:]
        gk = 32
        return _gat_sc(xl, xr, att.reshape(hc), bias, s_src, s_dst, ws,
                       n, npw, heads, cdim, gk)

    h = gat_layer(x, bn0_g, bn0_b, W1l, W1r, a1, b1, 2, 512, relu=False)
    h = gat_layer(h, bn1_g, bn1_b, W2l, W2r, a2, b2, 1, 512, relu=True)
    h = gat_layer(h, bn2_g, bn2_b, W3l, W3r, a3, b3, 1, 512, relu=True)
    h = gat_layer(h, bn3_g, bn3_b, W4l, W4r, a4, b4, 1, 512, relu=True)

    ones = jnp.ones((h.shape[1],), jnp.float32)
    zeros = jnp.zeros((h.shape[1],), jnp.float32)
    yp = _affine_matmul(h, lin_W, ones, zeros, lin_b, relu=True, sig=True)
    y_pred = yp[:, 0]

    idx = jnp.nonzero(train_mask, size=train_mask.shape[0], fill_value=0)[0]
    return (y_pred[idx], y[idx])
